# baseline (speedup 1.0000x reference)
import functools

import jax
import jax.numpy as jnp
from jax import lax
from jax.experimental import pallas as pl
from jax.experimental.pallas import tpu as pltpu
from jax.experimental.pallas import tpu_sc as plsc

_FREEZE_RATIO = 0.9
_NUM_CORES = 2
_NUM_SUBCORES = 16
_NW = _NUM_CORES * _NUM_SUBCORES
_LANES = 16
_GROUP = 8
_CHUNK_GROUPS = 4


@functools.cache
def _build(batch, seq, d):
    total = batch * seq
    target = int(total * _FREEZE_RATIO)
    chunk_rows = _CHUNK_GROUPS * _GROUP

    zg = target // _GROUP
    rem = target % _GROUP
    cg0 = zg + (1 if rem else 0)
    cg_n = total // _GROUP - cg0

    zgpw = -(-zg // _NW)
    cgpw = -(-cg_n // _NW)
    nzc = -(-zgpw // _CHUNK_GROUPS)
    ncc = -(-cgpw // _CHUNK_GROUPS)

    assert d % _LANES == 0 and total % _GROUP == 0
    assert zg >= _CHUNK_GROUPS and cg_n >= _CHUNK_GROUPS
    assert 4 * d * (chunk_rows * (1 + ncc) + _GROUP) <= 500_000

    def body(*refs):
        zsrc_ref, in_ref, out_ref, cnt_ref = refs[:4]
        zbuf = refs[4]
        dbufs = refs[5:5 + ncc]
        bbuf = refs[5 + ncc]
        cntbuf = refs[6 + ncc]
        wsem = refs[7 + ncc]
        bsem = refs[8 + ncc]
        ssem = refs[9 + ncc]
        lsems = refs[10 + ncc:10 + 2 * ncc]

        w = lax.axis_index("s") * _NUM_CORES + lax.axis_index("c")

        zs = jnp.minimum(w * zgpw, zg)
        ze = jnp.minimum(zs + zgpw, zg)
        cs = jnp.minimum(w * cgpw, cg_n)
        ce = jnp.minimum(cs + cgpw, cg_n)

        def zrow(i):
            g = jnp.maximum(0, jnp.minimum(zs + i * _CHUNK_GROUPS,
                                           ze - _CHUNK_GROUPS))
            return pl.multiple_of(g * _GROUP, _GROUP)

        def crow(i):
            g = jnp.maximum(0, jnp.minimum(cs + i * _CHUNK_GROUPS,
                                           ce - _CHUNK_GROUPS))
            return pl.multiple_of((cg0 + g) * _GROUP, _GROUP)

        def zero_dma(i):
            return pltpu.make_async_copy(
                zbuf, out_ref.at[pl.ds(zrow(i), chunk_rows)], wsem)

        def load_dma(i):
            return pltpu.make_async_copy(
                in_ref.at[pl.ds(crow(i), chunk_rows)], dbufs[i], lsems[i])

        def store_dma(i):
            return pltpu.make_async_copy(
                dbufs[i], out_ref.at[pl.ds(crow(i), chunk_rows)], wsem)

        brow = zg * _GROUP

        def boundary_load():
            return pltpu.make_async_copy(
                in_ref.at[pl.ds(brow, _GROUP)], bbuf, bsem)

        def boundary_store():
            return pltpu.make_async_copy(
                bbuf, out_ref.at[pl.ds(brow, _GROUP)], wsem)

        stage = pltpu.make_async_copy(
            zsrc_ref.at[pl.ds(pl.multiple_of(w * chunk_rows, _GROUP),
                              chunk_rows)], zbuf, ssem)
        stage.start()
        if rem:
            @pl.when(w == _NW - 1)
            def _():
                boundary_load().start()
        for i in range(ncc):
            load_dma(i).start()
        stage.wait()

        def zstart(i, carry):
            zero_dma(i).start()
            return carry

        lax.fori_loop(0, nzc, zstart, 0)

        @pl.when(w == 0)
        def _():
            cntbuf[...] = jnp.full((_LANES,), target, jnp.int32)
            pltpu.make_async_copy(cntbuf, cnt_ref, wsem).start()

        if rem:
            @pl.when(w == _NW - 1)
            def _():
                boundary_load().wait()
                zeros16 = jnp.zeros((_LANES,), jnp.float32)

                def bzero(i, carry):
                    bbuf[i // (d // _LANES),
                         pl.ds((i % (d // _LANES)) * _LANES, _LANES)] = zeros16
                    return carry

                lax.fori_loop(0, rem * (d // _LANES), bzero, 0)
                boundary_store().start()

        for i in range(ncc):
            load_dma(i).wait()
            store_dma(i).start()

        def zdrain(i, carry):
            zero_dma(i).wait()
            return carry

        lax.fori_loop(0, nzc, zdrain, 0)
        for i in range(ncc):
            store_dma(i).wait()
        if rem:
            @pl.when(w == _NW - 1)
            def _():
                boundary_store().wait()

        @pl.when(w == 0)
        def _():
            pltpu.make_async_copy(cntbuf, cnt_ref, wsem).wait()

    mesh = plsc.VectorSubcoreMesh(
        core_axis_name="c", subcore_axis_name="s",
        num_cores=_NUM_CORES, num_subcores=_NUM_SUBCORES)

    return pl.kernel(
        body,
        out_type=(
            jax.ShapeDtypeStruct((total, d), jnp.float32),
            jax.ShapeDtypeStruct((_LANES,), jnp.int32),
        ),
        mesh=mesh,
        scratch_types=(
            [pltpu.VMEM((chunk_rows, d), jnp.float32)]
            + [pltpu.VMEM((chunk_rows, d), jnp.float32)] * ncc
            + [pltpu.VMEM((_GROUP, d), jnp.float32)]
            + [pltpu.VMEM((_LANES,), jnp.int32)]
            + [pltpu.SemaphoreType.DMA]
            + [pltpu.SemaphoreType.DMA]
            + [pltpu.SemaphoreType.DMA]
            + [pltpu.SemaphoreType.DMA] * ncc
        ),
    )


def kernel(tokens):
    b, s, d = tokens.shape
    zsrc = jnp.zeros((_NW * _CHUNK_GROUPS * _GROUP, d), jnp.float32)
    out2d, cnt = _build(b, s, d)(zsrc, tokens.reshape(b * s, d))
    return out2d.reshape(b, s, d), cnt[0]


# speedup vs baseline: 1.0548x; 1.0548x over previous
import functools

import jax
import jax.numpy as jnp
from jax import lax
from jax.experimental import pallas as pl
from jax.experimental.pallas import tpu as pltpu
from jax.experimental.pallas import tpu_sc as plsc

_FREEZE_RATIO = 0.9
_NUM_CORES = 2
_NUM_SUBCORES = 16
_NW = _NUM_CORES * _NUM_SUBCORES
_LANES = 16
_GROUP = 8
_CHUNK_GROUPS = 4


@functools.cache
def _build(batch, seq, d):
    total = batch * seq
    target = int(total * _FREEZE_RATIO)
    chunk_rows = _CHUNK_GROUPS * _GROUP

    zg = target // _GROUP
    rem = target % _GROUP
    cg0 = zg + (1 if rem else 0)
    cg_n = total // _GROUP - cg0

    zgpw = -(-zg // _NW)
    cgpw = -(-cg_n // _NW)
    nzc = -(-zgpw // _CHUNK_GROUPS)
    ncc = -(-cgpw // _CHUNK_GROUPS)

    assert d % _LANES == 0 and total % _GROUP == 0
    assert zg >= _CHUNK_GROUPS and cg_n >= _CHUNK_GROUPS
    assert 4 * d * (chunk_rows * (1 + ncc) + _GROUP) <= 500_000

    def body(*refs):
        in_ref, out_ref, cnt_ref = refs[:3]
        zbuf = refs[3]
        dbufs = refs[4:4 + ncc]
        bbuf = refs[4 + ncc]
        cntbuf = refs[5 + ncc]
        wsem = refs[6 + ncc]
        bsem = refs[7 + ncc]
        lsems = refs[8 + ncc:8 + 2 * ncc]

        w = lax.axis_index("s") * _NUM_CORES + lax.axis_index("c")

        zs = jnp.minimum(w * zgpw, zg)
        ze = jnp.minimum(zs + zgpw, zg)
        cs = jnp.minimum(w * cgpw, cg_n)
        ce = jnp.minimum(cs + cgpw, cg_n)

        def zrow(i):
            g = jnp.maximum(0, jnp.minimum(zs + i * _CHUNK_GROUPS,
                                           ze - _CHUNK_GROUPS))
            return pl.multiple_of(g * _GROUP, _GROUP)

        def crow(i):
            g = jnp.maximum(0, jnp.minimum(cs + i * _CHUNK_GROUPS,
                                           ce - _CHUNK_GROUPS))
            return pl.multiple_of((cg0 + g) * _GROUP, _GROUP)

        def zero_dma(i):
            return pltpu.make_async_copy(
                zbuf, out_ref.at[pl.ds(zrow(i), chunk_rows)], wsem)

        def load_dma(i):
            return pltpu.make_async_copy(
                in_ref.at[pl.ds(crow(i), chunk_rows)], dbufs[i], lsems[i])

        def store_dma(i):
            return pltpu.make_async_copy(
                dbufs[i], out_ref.at[pl.ds(crow(i), chunk_rows)], wsem)

        brow = zg * _GROUP

        def boundary_load():
            return pltpu.make_async_copy(
                in_ref.at[pl.ds(brow, _GROUP)], bbuf, bsem)

        def boundary_store():
            return pltpu.make_async_copy(
                bbuf, out_ref.at[pl.ds(brow, _GROUP)], wsem)

        if rem:
            @pl.when(w == _NW - 1)
            def _():
                boundary_load().start()
        for i in range(ncc):
            load_dma(i).start()

        zeros16 = jnp.zeros((_LANES,), jnp.float32)
        unroll = 16
        assert d % (_LANES * unroll) == 0

        def fill(i, carry):
            r = i // (d // (_LANES * unroll))
            c = i % (d // (_LANES * unroll))
            for u in range(unroll):
                zbuf[r, pl.ds((c * unroll + u) * _LANES, _LANES)] = zeros16
            return carry

        lax.fori_loop(0, chunk_rows * d // (_LANES * unroll), fill, 0)

        def zstart(i, carry):
            zero_dma(i).start()
            return carry

        lax.fori_loop(0, nzc, zstart, 0)

        @pl.when(w == 0)
        def _():
            cntbuf[...] = jnp.full((_LANES,), target, jnp.int32)
            pltpu.make_async_copy(cntbuf, cnt_ref, wsem).start()

        if rem:
            @pl.when(w == _NW - 1)
            def _():
                boundary_load().wait()
                zeros16 = jnp.zeros((_LANES,), jnp.float32)

                def bzero(i, carry):
                    bbuf[i // (d // _LANES),
                         pl.ds((i % (d // _LANES)) * _LANES, _LANES)] = zeros16
                    return carry

                lax.fori_loop(0, rem * (d // _LANES), bzero, 0)
                boundary_store().start()

        for i in range(ncc):
            load_dma(i).wait()
            store_dma(i).start()

        def zdrain(i, carry):
            zero_dma(i).wait()
            return carry

        lax.fori_loop(0, nzc, zdrain, 0)
        for i in range(ncc):
            store_dma(i).wait()
        if rem:
            @pl.when(w == _NW - 1)
            def _():
                boundary_store().wait()

        @pl.when(w == 0)
        def _():
            pltpu.make_async_copy(cntbuf, cnt_ref, wsem).wait()

    mesh = plsc.VectorSubcoreMesh(
        core_axis_name="c", subcore_axis_name="s",
        num_cores=_NUM_CORES, num_subcores=_NUM_SUBCORES)

    return pl.kernel(
        body,
        out_type=(
            jax.ShapeDtypeStruct((total, d), jnp.float32),
            jax.ShapeDtypeStruct((_LANES,), jnp.int32),
        ),
        mesh=mesh,
        scratch_types=(
            [pltpu.VMEM((chunk_rows, d), jnp.float32)]
            + [pltpu.VMEM((chunk_rows, d), jnp.float32)] * ncc
            + [pltpu.VMEM((_GROUP, d), jnp.float32)]
            + [pltpu.VMEM((_LANES,), jnp.int32)]
            + [pltpu.SemaphoreType.DMA]
            + [pltpu.SemaphoreType.DMA]
            + [pltpu.SemaphoreType.DMA] * ncc
        ),
    )


def kernel(tokens):
    b, s, d = tokens.shape
    out2d, cnt = _build(b, s, d)(tokens.reshape(b * s, d))
    return out2d.reshape(b, s, d), cnt[0]
